# baseline (speedup 1.0000x reference)
import functools

import jax
import jax.numpy as jnp
from jax import lax
from jax.experimental import pallas as pl
from jax.experimental.pallas import tpu as pltpu
from jax.experimental.pallas import tpu_sc as plsc

S_NUM = 4096
E_NUM = 2048
K_NUM = 512
D = 64
N = S_NUM + E_NUM + K_NUM
NE = 200000
B = 16384
SSL_TEMP = 0.8
SSL_WEIGHT = 0.01

NC = 2
NS = 16

EBLK = 128
NBLK = 100
NE_PAD = NS * NBLK * EBLK
ROWS_PT = N // NS


def _sc_mesh():
    return plsc.VectorSubcoreMesh(core_axis_name="c", subcore_axis_name="s")


EPT = NBLK * EBLK
REAL_TAIL = NE - (NS - 1) * EPT
PADN = NE_PAD - NE


def _spmm_sc(rows_h, cols4, vals4, pad_c, pad_v, x0_h):
    out_t = [jax.ShapeDtypeStruct((4, N, D), jnp.float32),
             jax.ShapeDtypeStruct((4 * N, D), jnp.float32),
             jax.ShapeDtypeStruct((4 * N, D), jnp.float32)]

    @functools.partial(
        pl.kernel,
        out_type=out_t,
        mesh=_sc_mesh(),
        scratch_types=[
            pltpu.VMEM((NBLK, EBLK), jnp.int32),
            pltpu.VMEM((EPT,), jnp.int32),
            pltpu.VMEM((EPT,), jnp.float32),
            [pltpu.VMEM((EBLK, D), jnp.float32) for _ in range(4)],
            [pltpu.VMEM_SHARED((N, D), jnp.float32) for _ in range(2)],
            [pltpu.SemaphoreType.DMA for _ in range(4)],
            [pltpu.SemaphoreType.DMA for _ in range(4)],
        ],
        compiler_params=pltpu.CompilerParams(needs_layout_passes=False,
                                             use_tc_tiling_on_sc=False),
    )
    def spmm(rows_hr, c0, c1, c2, c3, v0, v1, v2, v3, padc, padv, x0_hr,
             sums_o, o0, o1, rowv, colv, valv, gb, accs, sg, ss):
        c = lax.axis_index("c")
        s = lax.axis_index("s")
        cols_r = [c0, c1, c2, c3]
        vals_r = [v0, v1, v2, v3]

        def _stage(col_h, val_h):
            @pl.when(s < NS - 1)
            def _():
                pltpu.sync_copy(col_h.at[pl.ds(s * EPT, EPT)], colv)
                pltpu.sync_copy(val_h.at[pl.ds(s * EPT, EPT)], valv)

            @pl.when(s == NS - 1)
            def _():
                tb = (NS - 1) * EPT
                pltpu.sync_copy(col_h.at[pl.ds(tb, REAL_TAIL)],
                                colv.at[pl.ds(0, REAL_TAIL)])
                pltpu.sync_copy(padc, colv.at[pl.ds(REAL_TAIL, PADN)])
                pltpu.sync_copy(val_h.at[pl.ds(tb, REAL_TAIL)],
                                valv.at[pl.ds(0, REAL_TAIL)])
                pltpu.sync_copy(padv, valv.at[pl.ds(REAL_TAIL, PADN)])

        def _zero(ref, rows):
            def _zb(i, _):
                for dd in range(D // 16):
                    ref[i, pl.ds(dd * 16, 16)] = jnp.zeros((16,), jnp.float32)
                return 0
            lax.fori_loop(0, rows, _zb, 0)

        chunks = []
        off = 0
        while off < ROWS_PT:
            chunks.append((off, min(EBLK, ROWS_PT - off)))
            off += EBLK

        for k in range(2):
            chain = 2 * k + c
            pltpu.sync_copy(rows_hr.at[chain, s], rowv)

            @pl.when(c == 0)
            def _():
                _stage(cols_r[2 * k], vals_r[2 * k])

            @pl.when(c == 1)
            def _():
                _stage(cols_r[2 * k + 1], vals_r[2 * k + 1])

            coff = chain * N

            def _off(i, _):
                sl = pl.ds(i * 16, 16)
                colv[sl] = colv[sl] + coff
                return 0

            for layer in range(3):
                if layer == 1:
                    lax.fori_loop(0, EPT // 16, _off, 0)
                src = x0_hr if layer == 0 else (o0 if layer == 1 else o1)
                acc = accs[layer % 2]

                if layer < 2:
                    _zero(gb[0], EBLK)
                    for co, cn in chunks:
                        pltpu.sync_copy(
                            gb[0].at[pl.ds(0, cn)],
                            acc.at[pl.ds(s * ROWS_PT + co, cn)])
                plsc.subcore_barrier()

                def _scale(g, j):
                    @plsc.parallel_loop(0, EBLK, 1, unroll=8)
                    def body(e):
                        ee = jnp.full((16,), j * EBLK + e, jnp.int32)
                        v = plsc.load_gather(valv, [ee])
                        for dd in range(D // 16):
                            sl = pl.ds(dd * 16, 16)
                            g[e, sl] = g[e, sl] * v

                def _cidx(j):
                    return colv.at[pl.ds(j * EBLK, EBLK)]

                def _gwait(b, j):
                    pltpu.make_async_copy(src.at[_cidx(j)], gb[b], sg[b]).wait()

                def _swait(b, j):
                    pltpu.make_async_copy(gb[b], acc.at[rowv.at[j]], ss[b]).wait()

                for b in range(2):
                    pltpu.async_copy(src.at[_cidx(b)], gb[b], sg[b])

                def _blk4(i, _):
                    for k in range(4):
                        j = 4 * i + k
                        _gwait(k, j)
                        _scale(gb[k], j)
                        pltpu.async_copy(gb[k], acc.at[rowv.at[j]], ss[k],
                                         add=True)
                        kp = (k + 2) % 4

                        @pl.when(j < NBLK - 2)
                        def _():
                            @pl.when(j >= 2)
                            def _():
                                _swait(kp, j)
                            pltpu.async_copy(src.at[_cidx(j + 2)], gb[kp],
                                             sg[kp])
                    return 0
                lax.fori_loop(0, NBLK // 4, _blk4, 0)
                for b in range(4):
                    _swait(b, 0)
                plsc.subcore_barrier()

                if layer < 2:
                    dst = o0 if layer == 0 else o1
                    pltpu.sync_copy(
                        acc.at[pl.ds(s * ROWS_PT, ROWS_PT)],
                        dst.at[pl.ds(chain * N + s * ROWS_PT, ROWS_PT)])

            for co, cn in chunks:
                base = s * ROWS_PT + co
                pltpu.sync_copy(accs[0].at[pl.ds(base, cn)],
                                gb[0].at[pl.ds(0, cn)])
                pltpu.sync_copy(accs[1].at[pl.ds(base, cn)],
                                gb[1].at[pl.ds(0, cn)])

                def _add(i, _):
                    for dd in range(D // 16):
                        sl = pl.ds(dd * 16, 16)
                        gb[0][i, sl] = gb[0][i, sl] + gb[1][i, sl]
                    return 0
                lax.fori_loop(0, cn, _add, 0)
                pltpu.sync_copy(gb[0].at[pl.ds(0, cn)],
                                sums_o.at[chain, pl.ds(base, cn)])

    return spmm(rows_h, *cols4, *vals4, pad_c, pad_v, x0_h)


def _mix_tc(x0, sums, concat_W, concat_b):
    BR = 832
    grid = N // BR

    def body(x0_r, s_r, w_r, b_r, of_r, off_r):
        w = w_r[...]
        wa = w[:, :D]
        wb = w[:, D:]
        b3 = 3.0 * b_r[...]
        x0b = x0_r[...]

        def mix(ir, iw):
            acc = lax.dot_general(s_r[ir], wa, (((1,), (1,)), ((), ())),
                                  preferred_element_type=jnp.float32)
            acc = acc + lax.dot_general(s_r[iw], wb, (((1,), (1,)), ((), ())),
                                        preferred_element_type=jnp.float32)
            return 0.25 * (x0b + acc + b3)

        of_r[...] = mix(0, 1)
        off_r[...] = mix(2, 3)

    of, off = pl.pallas_call(
        body,
        grid=(grid,),
        in_specs=[
            pl.BlockSpec((BR, D), lambda i: (i, 0)),
            pl.BlockSpec((4, BR, D), lambda i: (0, i, 0)),
            pl.BlockSpec((D, 2 * D), lambda i: (0, 0)),
            pl.BlockSpec((1, D), lambda i: (0, 0)),
        ],
        out_specs=[
            pl.BlockSpec((BR, D), lambda i: (i, 0)),
            pl.BlockSpec((BR, D), lambda i: (i, 0)),
        ],
        out_shape=[
            jax.ShapeDtypeStruct((N, D), jnp.float32),
            jax.ShapeDtypeStruct((N, D), jnp.float32),
        ],
    )(x0, sums, concat_W, concat_b.reshape(1, D))
    return of, off


def _linear_tc(x, w, b):
    m = x.shape[0]
    k = w.shape[0]
    br = min(m, 512)

    def body(x_r, w_r, b_r, o_r):
        o_r[...] = lax.dot_general(x_r[...], w_r[...], (((1,), (1,)), ((), ())),
                                   preferred_element_type=jnp.float32) + b_r[...]

    return pl.pallas_call(
        body,
        grid=(m // br,),
        in_specs=[
            pl.BlockSpec((br, D), lambda i: (i, 0)),
            pl.BlockSpec((k, D), lambda i: (0, 0)),
            pl.BlockSpec((1, k), lambda i: (0, 0)),
        ],
        out_specs=pl.BlockSpec((br, k), lambda i: (i, 0)),
        out_shape=jax.ShapeDtypeStruct((m, k), jnp.float32),
    )(x, w, b.reshape(1, k))


def _nce_sum_tc(a, bm):
    m = a.shape[0]
    br = 512

    def body(a_r, b_r, o_r):
        i = pl.program_id(0)
        av = a_r[...]
        bfull = b_r[...]
        scores = lax.dot_general(av, bfull, (((1,), (1,)), ((), ())),
                                 preferred_element_type=jnp.float32) / SSL_TEMP
        mx = jnp.max(scores, axis=1, keepdims=True)
        lse = mx[:, 0] + jnp.log(jnp.sum(jnp.exp(scores - mx), axis=1))
        bdiag = b_r[pl.ds(i * br, br), :]
        pos = jnp.sum(av * bdiag, axis=1) / SSL_TEMP
        contrib = jnp.sum(lse - pos)

        @pl.when(i == 0)
        def _():
            o_r[...] = jnp.zeros((1, 1), jnp.float32)

        o_r[...] += contrib.reshape(1, 1)

    return pl.pallas_call(
        body,
        grid=(m // br,),
        in_specs=[
            pl.BlockSpec((br, D), lambda i: (i, 0)),
            pl.BlockSpec((m, D), lambda i: (0, 0)),
        ],
        out_specs=pl.BlockSpec((1, 1), lambda i: (0, 0)),
        out_shape=jax.ShapeDtypeStruct((1, 1), jnp.float32),
    )(a, bm)


def _gather_tables_sc(tables, idxs, out_widths, tc_tiled, IDB, JB):
    nt = len(tables)
    out_t = [jax.ShapeDtypeStruct((B, w), jnp.float32) for w in out_widths]

    @functools.partial(
        pl.kernel,
        out_type=out_t,
        mesh=_sc_mesh(),
        scratch_types=[
            [pltpu.VMEM((JB, IDB), jnp.int32) for _ in range(nt)],
            [[pltpu.VMEM((IDB, w), jnp.float32) for _ in range(2)]
             for w in out_widths],
            [[pltpu.SemaphoreType.DMA for _ in range(2)] for _ in range(nt)],
            [[pltpu.SemaphoreType.DMA for _ in range(2)] for _ in range(nt)],
        ],
        compiler_params=pltpu.CompilerParams(needs_layout_passes=False,
                                             use_tc_tiling_on_sc=tc_tiled),
    )
    def gk(*refs):
        tabs_h = refs[:nt]
        idx_h = refs[nt:2 * nt]
        dst_h = refs[2 * nt:3 * nt]
        idxv, bufs, gsem, wsem = refs[3 * nt:3 * nt + 4]
        c = lax.axis_index("c")
        s = lax.axis_index("s")
        wid = s * NC + c
        base = wid * (JB * IDB)
        for t in range(nt):
            pltpu.sync_copy(idx_h[t].at[wid], idxv[t])
        for t in range(nt):
            src_h, iv, dst = tabs_h[t], idxv[t], dst_h[t]
            pltpu.async_copy(src_h.at[iv.at[0]], bufs[t][0], gsem[t][0])
            for j in range(JB):
                p = j % 2
                o = base + j * IDB
                pltpu.make_async_copy(src_h.at[iv.at[j]], bufs[t][p],
                                      gsem[t][p]).wait()
                if j + 1 < JB:
                    pn = (j + 1) % 2
                    if j >= 1:
                        pltpu.make_async_copy(
                            bufs[t][pn], dst.at[pl.ds(o, IDB)],
                            wsem[t][pn]).wait()
                    pltpu.async_copy(src_h.at[iv.at[j + 1]], bufs[t][pn],
                                     gsem[t][pn])
                pltpu.async_copy(bufs[t][p], dst.at[pl.ds(o, IDB)],
                                 wsem[t][p])
            for p in range(2):
                pltpu.make_async_copy(bufs[t][p], dst.at[pl.ds(base, IDB)],
                                      wsem[t][p]).wait()

    return gk(*tables, *idxs)


def kernel(student_id, exercise_id, q_mask, right_idx, right_val,
           wrong_idx, wrong_val, right_flip_idx, right_flip_val,
           wrong_flip_idx, wrong_flip_val, stu_emb, exer_emb, know_emb,
           disc_emb, ki_emb, concat_W, concat_b, Wts, bts, Wte, bte,
           Wtk, btk):
    f32 = jnp.float32
    x0 = jnp.concatenate([stu_emb, exer_emb, know_emb], axis=0)

    pad_idx = (jnp.arange(PADN, dtype=jnp.int32) % N)
    pad_vals = jnp.zeros((PADN,), f32)

    chains = [(right_idx, right_val), (wrong_idx, wrong_val),
              (right_flip_idx, right_flip_val), (wrong_flip_idx, wrong_flip_val)]
    rows_h = jnp.stack(
        [jnp.concatenate([i[0].astype(jnp.int32), pad_idx]) for i, _ in chains]
    ).reshape(4, NS, NBLK, EBLK)
    cols4 = [i[1].astype(jnp.int32) for i, _ in chains]
    vals4 = [v.astype(f32) for _, v in chains]

    sums, _, _ = _spmm_sc(rows_h, cols4, vals4, pad_idx, pad_vals, x0)

    out_f, out_ff = _mix_tc(x0, sums, concat_W, concat_b)
    s_f = out_f[:S_NUM]
    e_f = out_f[S_NUM:S_NUM + E_NUM]
    k_f = out_f[S_NUM + E_NUM:]
    s_ff = out_ff[:S_NUM]
    e_ff = out_ff[S_NUM:S_NUM + E_NUM]

    ps = _linear_tc(s_f, Wts, bts)
    pe = _linear_tc(e_f, Wte, bte)
    knowledge_ts = _linear_tc(k_f, Wtk, btk)

    ss = _nce_sum_tc(s_f, s_ff)
    se = _nce_sum_tc(e_f, e_ff)
    extra_loss = (SSL_WEIGHT * (ss[0, 0] / S_NUM + se[0, 0] / E_NUM)).astype(f32)

    sid32 = student_id.astype(jnp.int32).reshape(32, 16, 32)
    eid32 = exercise_id.astype(jnp.int32).reshape(32, 16, 32)
    tke = jnp.concatenate(
        [ki_emb, disc_emb, jnp.zeros((E_NUM, 63), f32)], axis=1)
    student_ts, diff_ts, tke_ts = _gather_tables_sc(
        [ps, pe, tke], [sid32, eid32, eid32],
        [K_NUM, K_NUM, 2 * D], True, 32, 16)
    ki_ts = tke_ts[:, :D]
    disc_ts = tke_ts[:, D:D + 1]

    return (student_ts, diff_ts, disc_ts, knowledge_ts, extra_loss, ki_ts)


# speedup vs baseline: 1.0653x; 1.0653x over previous
import functools

import jax
import jax.numpy as jnp
from jax import lax
from jax.experimental import pallas as pl
from jax.experimental.pallas import tpu as pltpu
from jax.experimental.pallas import tpu_sc as plsc

S_NUM = 4096
E_NUM = 2048
K_NUM = 512
D = 64
N = S_NUM + E_NUM + K_NUM
NE = 200000
B = 16384
SSL_TEMP = 0.8
SSL_WEIGHT = 0.01

NC = 2
NS = 16

EBLK = 128
NBLK = 100
NE_PAD = NS * NBLK * EBLK
ROWS_PT = N // NS


def _sc_mesh():
    return plsc.VectorSubcoreMesh(core_axis_name="c", subcore_axis_name="s")


def _spmm_sc(rows_h, cols_h, vals_h, x0_h):
    out_t = [jax.ShapeDtypeStruct((4, N, D), jnp.float32),
             jax.ShapeDtypeStruct((4 * N, D), jnp.float32),
             jax.ShapeDtypeStruct((4 * N, D), jnp.float32)]

    @functools.partial(
        pl.kernel,
        out_type=out_t,
        mesh=_sc_mesh(),
        scratch_types=[
            pltpu.VMEM((NBLK, EBLK), jnp.int32),
            pltpu.VMEM((NBLK, EBLK), jnp.int32),
            pltpu.VMEM((NBLK * EBLK,), jnp.float32),
            [pltpu.VMEM((EBLK, D), jnp.float32) for _ in range(4)],
            [pltpu.VMEM_SHARED((N, D), jnp.float32) for _ in range(2)],
            [pltpu.SemaphoreType.DMA for _ in range(4)],
            [pltpu.SemaphoreType.DMA for _ in range(4)],
        ],
        compiler_params=pltpu.CompilerParams(needs_layout_passes=False,
                                             use_tc_tiling_on_sc=False),
    )
    def spmm(rows_hr, cols_hr, vals_hr, x0_hr,
             sums_o, o0, o1, rowv, colv, valv, gb, accs, sg, ss):
        c = lax.axis_index("c")
        s = lax.axis_index("s")

        def _zero(ref, rows):
            def _zb(i, _):
                for dd in range(D // 16):
                    ref[i, pl.ds(dd * 16, 16)] = jnp.zeros((16,), jnp.float32)
                return 0
            lax.fori_loop(0, rows, _zb, 0)

        chunks = []
        off = 0
        while off < ROWS_PT:
            chunks.append((off, min(EBLK, ROWS_PT - off)))
            off += EBLK

        for k in range(2):
            chain = 2 * k + c
            pltpu.sync_copy(rows_hr.at[chain, s], rowv)
            pltpu.sync_copy(cols_hr.at[chain, s], colv)
            pltpu.sync_copy(vals_hr.at[chain, s], valv)

            coff = chain * N

            def _off(j, _):
                for t in range(EBLK // 16):
                    sl = pl.ds(t * 16, 16)
                    colv[j, sl] = colv[j, sl] + coff
                return 0

            for layer in range(3):
                if layer == 1:
                    lax.fori_loop(0, NBLK, _off, 0)
                src = x0_hr if layer == 0 else (o0 if layer == 1 else o1)
                acc = accs[layer % 2]

                if layer < 2:
                    _zero(gb[0], EBLK)
                    for co, cn in chunks:
                        pltpu.sync_copy(
                            gb[0].at[pl.ds(0, cn)],
                            acc.at[pl.ds(s * ROWS_PT + co, cn)])
                plsc.subcore_barrier()

                def _scale(g, j):
                    @plsc.parallel_loop(0, EBLK, 1, unroll=4)
                    def body(e):
                        ee = jnp.full((16,), j * EBLK + e, jnp.int32)
                        v = plsc.load_gather(valv, [ee])
                        for dd in range(D // 16):
                            sl = pl.ds(dd * 16, 16)
                            g[e, sl] = g[e, sl] * v

                def _cidx(j):
                    return colv.at[j]

                def _gwait(b, j):
                    pltpu.make_async_copy(src.at[_cidx(j)], gb[b], sg[b]).wait()

                def _swait(b, j):
                    pltpu.make_async_copy(gb[b], acc.at[rowv.at[j]], ss[b]).wait()

                for b in range(3):
                    pltpu.async_copy(src.at[_cidx(b)], gb[b], sg[b])

                def _blk4(i, _):
                    for k in range(4):
                        j = 4 * i + k
                        _gwait(k, j)
                        _scale(gb[k], j)
                        pltpu.async_copy(gb[k], acc.at[rowv.at[j]], ss[k],
                                         add=True)
                        kp = (k + 3) % 4

                        @pl.when(j < NBLK - 3)
                        def _():
                            @pl.when(j >= 1)
                            def _():
                                _swait(kp, j)
                            pltpu.async_copy(src.at[_cidx(j + 3)], gb[kp],
                                             sg[kp])
                    return 0
                lax.fori_loop(0, NBLK // 4, _blk4, 0)
                for b in range(4):
                    _swait(b, 0)
                plsc.subcore_barrier()

                if layer < 2:
                    dst = o0 if layer == 0 else o1
                    pltpu.sync_copy(
                        acc.at[pl.ds(s * ROWS_PT, ROWS_PT)],
                        dst.at[pl.ds(chain * N + s * ROWS_PT, ROWS_PT)])

            for co, cn in chunks:
                base = s * ROWS_PT + co
                pltpu.sync_copy(accs[0].at[pl.ds(base, cn)],
                                gb[0].at[pl.ds(0, cn)])
                pltpu.sync_copy(accs[1].at[pl.ds(base, cn)],
                                gb[1].at[pl.ds(0, cn)])

                def _add(i, _):
                    for dd in range(D // 16):
                        sl = pl.ds(dd * 16, 16)
                        gb[0][i, sl] = gb[0][i, sl] + gb[1][i, sl]
                    return 0
                lax.fori_loop(0, cn, _add, 0)
                pltpu.sync_copy(gb[0].at[pl.ds(0, cn)],
                                sums_o.at[chain, pl.ds(base, cn)])

    return spmm(rows_h, cols_h, vals_h, x0_h)


def _mix_tc(x0, sums, concat_W, concat_b):
    BR = 832
    grid = N // BR

    def body(x0_r, s_r, w_r, b_r, of_r, off_r):
        w = w_r[...]
        wa = w[:, :D]
        wb = w[:, D:]
        b3 = 3.0 * b_r[...]
        x0b = x0_r[...]

        def mix(ir, iw):
            acc = lax.dot_general(s_r[ir], wa, (((1,), (1,)), ((), ())),
                                  preferred_element_type=jnp.float32)
            acc = acc + lax.dot_general(s_r[iw], wb, (((1,), (1,)), ((), ())),
                                        preferred_element_type=jnp.float32)
            return 0.25 * (x0b + acc + b3)

        of_r[...] = mix(0, 1)
        off_r[...] = mix(2, 3)

    of, off = pl.pallas_call(
        body,
        grid=(grid,),
        in_specs=[
            pl.BlockSpec((BR, D), lambda i: (i, 0)),
            pl.BlockSpec((4, BR, D), lambda i: (0, i, 0)),
            pl.BlockSpec((D, 2 * D), lambda i: (0, 0)),
            pl.BlockSpec((1, D), lambda i: (0, 0)),
        ],
        out_specs=[
            pl.BlockSpec((BR, D), lambda i: (i, 0)),
            pl.BlockSpec((BR, D), lambda i: (i, 0)),
        ],
        out_shape=[
            jax.ShapeDtypeStruct((N, D), jnp.float32),
            jax.ShapeDtypeStruct((N, D), jnp.float32),
        ],
    )(x0, sums, concat_W, concat_b.reshape(1, D))
    return of, off


def _linear_tc(x, w, b):
    m = x.shape[0]
    k = w.shape[0]
    br = min(m, 512)

    def body(x_r, w_r, b_r, o_r):
        o_r[...] = lax.dot_general(x_r[...], w_r[...], (((1,), (1,)), ((), ())),
                                   preferred_element_type=jnp.float32) + b_r[...]

    return pl.pallas_call(
        body,
        grid=(m // br,),
        in_specs=[
            pl.BlockSpec((br, D), lambda i: (i, 0)),
            pl.BlockSpec((k, D), lambda i: (0, 0)),
            pl.BlockSpec((1, k), lambda i: (0, 0)),
        ],
        out_specs=pl.BlockSpec((br, k), lambda i: (i, 0)),
        out_shape=jax.ShapeDtypeStruct((m, k), jnp.float32),
    )(x, w, b.reshape(1, k))


def _nce_sum_tc(a, bm):
    m = a.shape[0]
    br = 512

    def body(a_r, b_r, o_r):
        i = pl.program_id(0)
        av = a_r[...]
        bfull = b_r[...]
        scores = lax.dot_general(av, bfull, (((1,), (1,)), ((), ())),
                                 preferred_element_type=jnp.float32) / SSL_TEMP
        mx = jnp.max(scores, axis=1, keepdims=True)
        lse = mx[:, 0] + jnp.log(jnp.sum(jnp.exp(scores - mx), axis=1))
        bdiag = b_r[pl.ds(i * br, br), :]
        pos = jnp.sum(av * bdiag, axis=1) / SSL_TEMP
        contrib = jnp.sum(lse - pos)

        @pl.when(i == 0)
        def _():
            o_r[...] = jnp.zeros((1, 1), jnp.float32)

        o_r[...] += contrib.reshape(1, 1)

    return pl.pallas_call(
        body,
        grid=(m // br,),
        in_specs=[
            pl.BlockSpec((br, D), lambda i: (i, 0)),
            pl.BlockSpec((m, D), lambda i: (0, 0)),
        ],
        out_specs=pl.BlockSpec((1, 1), lambda i: (0, 0)),
        out_shape=jax.ShapeDtypeStruct((1, 1), jnp.float32),
    )(a, bm)


def _gather_tables_sc(tables, idxs, out_widths, tc_tiled, IDB, JB):
    nt = len(tables)
    out_t = [jax.ShapeDtypeStruct((B, w), jnp.float32) for w in out_widths]

    @functools.partial(
        pl.kernel,
        out_type=out_t,
        mesh=_sc_mesh(),
        scratch_types=[
            [pltpu.VMEM((JB, IDB), jnp.int32) for _ in range(nt)],
            [[pltpu.VMEM((IDB, w), jnp.float32) for _ in range(2)]
             for w in out_widths],
            [[pltpu.SemaphoreType.DMA for _ in range(2)] for _ in range(nt)],
            [[pltpu.SemaphoreType.DMA for _ in range(2)] for _ in range(nt)],
        ],
        compiler_params=pltpu.CompilerParams(needs_layout_passes=False,
                                             use_tc_tiling_on_sc=tc_tiled),
    )
    def gk(*refs):
        tabs_h = refs[:nt]
        idx_h = refs[nt:2 * nt]
        dst_h = refs[2 * nt:3 * nt]
        idxv, bufs, gsem, wsem = refs[3 * nt:3 * nt + 4]
        c = lax.axis_index("c")
        s = lax.axis_index("s")
        wid = s * NC + c
        base = wid * (JB * IDB)
        for t in range(nt):
            pltpu.sync_copy(idx_h[t].at[wid], idxv[t])
        for t in range(nt):
            src_h, iv, dst = tabs_h[t], idxv[t], dst_h[t]
            pltpu.async_copy(src_h.at[iv.at[0]], bufs[t][0], gsem[t][0])
            for j in range(JB):
                p = j % 2
                o = base + j * IDB
                pltpu.make_async_copy(src_h.at[iv.at[j]], bufs[t][p],
                                      gsem[t][p]).wait()
                if j + 1 < JB:
                    pn = (j + 1) % 2
                    if j >= 1:
                        pltpu.make_async_copy(
                            bufs[t][pn], dst.at[pl.ds(o, IDB)],
                            wsem[t][pn]).wait()
                    pltpu.async_copy(src_h.at[iv.at[j + 1]], bufs[t][pn],
                                     gsem[t][pn])
                pltpu.async_copy(bufs[t][p], dst.at[pl.ds(o, IDB)],
                                 wsem[t][p])
            for p in range(2):
                pltpu.make_async_copy(bufs[t][p], dst.at[pl.ds(base, IDB)],
                                      wsem[t][p]).wait()

    return gk(*tables, *idxs)


def kernel(student_id, exercise_id, q_mask, right_idx, right_val,
           wrong_idx, wrong_val, right_flip_idx, right_flip_val,
           wrong_flip_idx, wrong_flip_val, stu_emb, exer_emb, know_emb,
           disc_emb, ki_emb, concat_W, concat_b, Wts, bts, Wte, bte,
           Wtk, btk):
    f32 = jnp.float32
    x0 = jnp.concatenate([stu_emb, exer_emb, know_emb], axis=0)

    pad = NE_PAD - NE
    pad_idx = (jnp.arange(pad, dtype=jnp.int32) % N)

    def prep(idx, val):
        r = jnp.concatenate([idx[0].astype(jnp.int32), pad_idx])
        cc = jnp.concatenate([idx[1].astype(jnp.int32), pad_idx])
        v = jnp.concatenate([val.astype(f32), jnp.zeros((pad,), f32)])
        return r, cc, v

    chains = [(right_idx, right_val), (wrong_idx, wrong_val),
              (right_flip_idx, right_flip_val), (wrong_flip_idx, wrong_flip_val)]
    rs, cs, vs = zip(*(prep(i, v) for i, v in chains))
    rows_h = jnp.stack(rs).reshape(4, NS, NBLK, EBLK)
    cols_h = jnp.stack(cs).reshape(4, NS, NBLK, EBLK)
    vals_h = jnp.stack(vs).reshape(4, NS, NBLK * EBLK)

    sums, _, _ = _spmm_sc(rows_h, cols_h, vals_h, x0)

    out_f, out_ff = _mix_tc(x0, sums, concat_W, concat_b)
    s_f = out_f[:S_NUM]
    e_f = out_f[S_NUM:S_NUM + E_NUM]
    k_f = out_f[S_NUM + E_NUM:]
    s_ff = out_ff[:S_NUM]
    e_ff = out_ff[S_NUM:S_NUM + E_NUM]

    ps = _linear_tc(s_f, Wts, bts)
    pe = _linear_tc(e_f, Wte, bte)
    knowledge_ts = _linear_tc(k_f, Wtk, btk)

    ss = _nce_sum_tc(s_f, s_ff)
    se = _nce_sum_tc(e_f, e_ff)
    extra_loss = (SSL_WEIGHT * (ss[0, 0] / S_NUM + se[0, 0] / E_NUM)).astype(f32)

    sid32 = student_id.astype(jnp.int32).reshape(32, 16, 32)
    eid32 = exercise_id.astype(jnp.int32).reshape(32, 16, 32)
    tke = jnp.concatenate(
        [ki_emb, disc_emb, jnp.zeros((E_NUM, 63), f32)], axis=1)
    student_ts, diff_ts, tke_ts = _gather_tables_sc(
        [ps, pe, tke], [sid32, eid32, eid32],
        [K_NUM, K_NUM, 2 * D], True, 32, 16)
    ki_ts = tke_ts[:, :D]
    disc_ts = tke_ts[:, D:D + 1]

    return (student_ts, diff_ts, disc_ts, knowledge_ts, extra_loss, ki_ts)


# speedup vs baseline: 1.0912x; 1.0243x over previous
import functools

import jax
import jax.numpy as jnp
from jax import lax
from jax.experimental import pallas as pl
from jax.experimental.pallas import tpu as pltpu
from jax.experimental.pallas import tpu_sc as plsc

S_NUM = 4096
E_NUM = 2048
K_NUM = 512
D = 64
N = S_NUM + E_NUM + K_NUM
NE = 200000
B = 16384
SSL_TEMP = 0.8
SSL_WEIGHT = 0.01

NC = 2
NS = 16

EBLK = 128
NBLK = 100
NE_PAD = NS * NBLK * EBLK
ROWS_PT = N // NS


def _sc_mesh():
    return plsc.VectorSubcoreMesh(core_axis_name="c", subcore_axis_name="s")


def _spmm_sc(rows_h, cols_h, vals_h, x0_h):
    out_t = [jax.ShapeDtypeStruct((4, N, D), jnp.float32),
             jax.ShapeDtypeStruct((4 * N, D), jnp.float32),
             jax.ShapeDtypeStruct((4 * N, D), jnp.float32)]

    @functools.partial(
        pl.kernel,
        out_type=out_t,
        mesh=_sc_mesh(),
        scratch_types=[
            pltpu.VMEM((NBLK, EBLK), jnp.int32),
            pltpu.VMEM((NBLK, EBLK), jnp.int32),
            pltpu.VMEM((NBLK * EBLK,), jnp.float32),
            [pltpu.VMEM((EBLK, D), jnp.float32) for _ in range(4)],
            [pltpu.VMEM_SHARED((N, D), jnp.float32) for _ in range(2)],
            [pltpu.SemaphoreType.DMA for _ in range(4)],
            [pltpu.SemaphoreType.DMA for _ in range(4)],
        ],
        compiler_params=pltpu.CompilerParams(needs_layout_passes=False,
                                             use_tc_tiling_on_sc=False),
    )
    def spmm(rows_hr, cols_hr, vals_hr, x0_hr,
             sums_o, o0, o1, rowv, colv, valv, gb, accs, sg, ss):
        c = lax.axis_index("c")
        s = lax.axis_index("s")

        def _zero(ref, rows):
            def _zb(i, _):
                for dd in range(D // 16):
                    ref[i, pl.ds(dd * 16, 16)] = jnp.zeros((16,), jnp.float32)
                return 0
            lax.fori_loop(0, rows, _zb, 0)

        chunks = []
        off = 0
        while off < ROWS_PT:
            chunks.append((off, min(EBLK, ROWS_PT - off)))
            off += EBLK

        for k in range(2):
            chain = 2 * k + c
            pltpu.sync_copy(rows_hr.at[chain, s], rowv)
            pltpu.sync_copy(cols_hr.at[chain, s], colv)
            pltpu.sync_copy(vals_hr.at[chain, s], valv)

            coff = chain * N

            def _off(j, _):
                for t in range(EBLK // 16):
                    sl = pl.ds(t * 16, 16)
                    colv[j, sl] = colv[j, sl] + coff
                return 0

            for layer in range(3):
                if layer == 1:
                    lax.fori_loop(0, NBLK, _off, 0)
                src = x0_hr if layer == 0 else (o0 if layer == 1 else o1)
                acc = accs[layer % 2]

                if layer < 2:
                    _zero(gb[0], EBLK)
                    for co, cn in chunks:
                        pltpu.sync_copy(
                            gb[0].at[pl.ds(0, cn)],
                            acc.at[pl.ds(s * ROWS_PT + co, cn)])
                plsc.subcore_barrier()

                def _scale(g, j):
                    @plsc.parallel_loop(0, EBLK, 1, unroll=4)
                    def body(e):
                        ee = jnp.full((16,), j * EBLK + e, jnp.int32)
                        v = plsc.load_gather(valv, [ee])
                        for dd in range(D // 16):
                            sl = pl.ds(dd * 16, 16)
                            g[e, sl] = g[e, sl] * v

                def _cidx(j):
                    return colv.at[j]

                def _gwait(b, j):
                    pltpu.make_async_copy(src.at[_cidx(j)], gb[b], sg[b]).wait()

                def _swait(b, j):
                    pltpu.make_async_copy(gb[b], acc.at[rowv.at[j]], ss[b]).wait()

                for b in range(3):
                    pltpu.async_copy(src.at[_cidx(b)], gb[b], sg[b])

                def _blk4(i, _):
                    for k in range(4):
                        j = 4 * i + k
                        _gwait(k, j)
                        _scale(gb[k], j)
                        pltpu.async_copy(gb[k], acc.at[rowv.at[j]], ss[k],
                                         add=True)
                        kp = (k + 3) % 4

                        @pl.when(j < NBLK - 3)
                        def _():
                            @pl.when(j >= 1)
                            def _():
                                _swait(kp, j)
                            pltpu.async_copy(src.at[_cidx(j + 3)], gb[kp],
                                             sg[kp])
                    return 0
                lax.fori_loop(0, NBLK // 4, _blk4, 0)
                for b in range(4):
                    _swait(b, 0)
                plsc.subcore_barrier()

                if layer < 2:
                    dst = o0 if layer == 0 else o1
                    pltpu.sync_copy(
                        acc.at[pl.ds(s * ROWS_PT, ROWS_PT)],
                        dst.at[pl.ds(chain * N + s * ROWS_PT, ROWS_PT)])

            for co, cn in chunks:
                base = s * ROWS_PT + co
                pltpu.sync_copy(accs[0].at[pl.ds(base, cn)],
                                gb[0].at[pl.ds(0, cn)])
                pltpu.sync_copy(accs[1].at[pl.ds(base, cn)],
                                gb[1].at[pl.ds(0, cn)])

                def _add(i, _):
                    for dd in range(D // 16):
                        sl = pl.ds(dd * 16, 16)
                        gb[0][i, sl] = gb[0][i, sl] + gb[1][i, sl]
                    return 0
                lax.fori_loop(0, cn, _add, 0)
                pltpu.sync_copy(gb[0].at[pl.ds(0, cn)],
                                sums_o.at[chain, pl.ds(base, cn)])

    return spmm(rows_h, cols_h, vals_h, x0_h)


def _mix_tc(x0, sums, concat_W, concat_b):
    BR = 832
    grid = N // BR

    def body(x0_r, s_r, w_r, b_r, of_r, off_r):
        w = w_r[...]
        wa = w[:, :D]
        wb = w[:, D:]
        b3 = 3.0 * b_r[...]
        x0b = x0_r[...]

        def mix(ir, iw):
            acc = lax.dot_general(s_r[ir], wa, (((1,), (1,)), ((), ())),
                                  preferred_element_type=jnp.float32)
            acc = acc + lax.dot_general(s_r[iw], wb, (((1,), (1,)), ((), ())),
                                        preferred_element_type=jnp.float32)
            return 0.25 * (x0b + acc + b3)

        of_r[...] = mix(0, 1)
        off_r[...] = mix(2, 3)

    of, off = pl.pallas_call(
        body,
        grid=(grid,),
        in_specs=[
            pl.BlockSpec((BR, D), lambda i: (i, 0)),
            pl.BlockSpec((4, BR, D), lambda i: (0, i, 0)),
            pl.BlockSpec((D, 2 * D), lambda i: (0, 0)),
            pl.BlockSpec((1, D), lambda i: (0, 0)),
        ],
        out_specs=[
            pl.BlockSpec((BR, D), lambda i: (i, 0)),
            pl.BlockSpec((BR, D), lambda i: (i, 0)),
        ],
        out_shape=[
            jax.ShapeDtypeStruct((N, D), jnp.float32),
            jax.ShapeDtypeStruct((N, D), jnp.float32),
        ],
    )(x0, sums, concat_W, concat_b.reshape(1, D))
    return of, off


def _linear_tc(x, w, b):
    m = x.shape[0]
    k = w.shape[0]
    br = min(m, 512)

    def body(x_r, w_r, b_r, o_r):
        o_r[...] = lax.dot_general(x_r[...], w_r[...], (((1,), (1,)), ((), ())),
                                   preferred_element_type=jnp.float32) + b_r[...]

    return pl.pallas_call(
        body,
        grid=(m // br,),
        in_specs=[
            pl.BlockSpec((br, D), lambda i: (i, 0)),
            pl.BlockSpec((k, D), lambda i: (0, 0)),
            pl.BlockSpec((1, k), lambda i: (0, 0)),
        ],
        out_specs=pl.BlockSpec((br, k), lambda i: (i, 0)),
        out_shape=jax.ShapeDtypeStruct((m, k), jnp.float32),
    )(x, w, b.reshape(1, k))


def _nce_sum_tc(a, bm):
    m = a.shape[0]
    br = 512

    def body(a_r, b_r, o_r):
        i = pl.program_id(0)
        av = a_r[...]
        bfull = b_r[...]
        scores = lax.dot_general(av, bfull, (((1,), (1,)), ((), ())),
                                 preferred_element_type=jnp.float32) / SSL_TEMP
        mx = jnp.max(scores, axis=1, keepdims=True)
        lse = mx[:, 0] + jnp.log(jnp.sum(jnp.exp(scores - mx), axis=1))
        bdiag = b_r[pl.ds(i * br, br), :]
        pos = jnp.sum(av * bdiag, axis=1) / SSL_TEMP
        contrib = jnp.sum(lse - pos)

        @pl.when(i == 0)
        def _():
            o_r[...] = jnp.zeros((1, 1), jnp.float32)

        o_r[...] += contrib.reshape(1, 1)

    return pl.pallas_call(
        body,
        grid=(m // br,),
        in_specs=[
            pl.BlockSpec((br, D), lambda i: (i, 0)),
            pl.BlockSpec((m, D), lambda i: (0, 0)),
        ],
        out_specs=pl.BlockSpec((1, 1), lambda i: (0, 0)),
        out_shape=jax.ShapeDtypeStruct((1, 1), jnp.float32),
    )(a, bm)


def _gather_tables_sc(tables, idxs, out_widths, tc_tiled, IDB, JB):
    nt = len(tables)
    out_t = [jax.ShapeDtypeStruct((B, w), jnp.float32) for w in out_widths]

    @functools.partial(
        pl.kernel,
        out_type=out_t,
        mesh=_sc_mesh(),
        scratch_types=[
            [pltpu.VMEM((JB, IDB), jnp.int32) for _ in range(nt)],
            [[pltpu.VMEM((IDB, w), jnp.float32) for _ in range(2)]
             for w in out_widths],
            [[pltpu.SemaphoreType.DMA for _ in range(2)] for _ in range(nt)],
            [[pltpu.SemaphoreType.DMA for _ in range(2)] for _ in range(nt)],
        ],
        compiler_params=pltpu.CompilerParams(needs_layout_passes=False,
                                             use_tc_tiling_on_sc=tc_tiled),
    )
    def gk(*refs):
        tabs_h = refs[:nt]
        idx_h = refs[nt:2 * nt]
        dst_h = refs[2 * nt:3 * nt]
        idxv, bufs, gsem, wsem = refs[3 * nt:3 * nt + 4]
        c = lax.axis_index("c")
        s = lax.axis_index("s")
        wid = s * NC + c
        base = wid * (JB * IDB)
        for t in range(nt):
            pltpu.sync_copy(idx_h[t].at[wid], idxv[t])
        for t in range(nt):
            src_h, iv, dst = tabs_h[t], idxv[t], dst_h[t]
            pltpu.async_copy(src_h.at[iv.at[0]], bufs[t][0], gsem[t][0])
            for j in range(JB):
                p = j % 2
                o = base + j * IDB
                pltpu.make_async_copy(src_h.at[iv.at[j]], bufs[t][p],
                                      gsem[t][p]).wait()
                if j + 1 < JB:
                    pn = (j + 1) % 2
                    if j >= 1:
                        pltpu.make_async_copy(
                            bufs[t][pn], dst.at[pl.ds(o, IDB)],
                            wsem[t][pn]).wait()
                    pltpu.async_copy(src_h.at[iv.at[j + 1]], bufs[t][pn],
                                     gsem[t][pn])
                pltpu.async_copy(bufs[t][p], dst.at[pl.ds(o, IDB)],
                                 wsem[t][p])
            for p in range(2):
                pltpu.make_async_copy(bufs[t][p], dst.at[pl.ds(base, IDB)],
                                      wsem[t][p]).wait()

    return gk(*tables, *idxs)


def kernel(student_id, exercise_id, q_mask, right_idx, right_val,
           wrong_idx, wrong_val, right_flip_idx, right_flip_val,
           wrong_flip_idx, wrong_flip_val, stu_emb, exer_emb, know_emb,
           disc_emb, ki_emb, concat_W, concat_b, Wts, bts, Wte, bte,
           Wtk, btk):
    f32 = jnp.float32
    x0 = jnp.concatenate([stu_emb, exer_emb, know_emb], axis=0)

    pad = NE_PAD - NE
    pad_idx = (jnp.arange(pad, dtype=jnp.int32) % N)

    def prep(idx, val):
        r = jnp.concatenate([idx[0].astype(jnp.int32), pad_idx])
        cc = jnp.concatenate([idx[1].astype(jnp.int32), pad_idx])
        v = jnp.concatenate([val.astype(f32), jnp.zeros((pad,), f32)])
        return r, cc, v

    chains = [(right_idx, right_val), (wrong_idx, wrong_val),
              (right_flip_idx, right_flip_val), (wrong_flip_idx, wrong_flip_val)]
    rs, cs, vs = zip(*(prep(i, v) for i, v in chains))
    rows_h = jnp.stack(rs).reshape(4, NS, NBLK, EBLK)
    cols_h = jnp.stack(cs).reshape(4, NS, NBLK, EBLK)
    vals_h = jnp.stack(vs).reshape(4, NS, NBLK * EBLK)

    sums, _, _ = _spmm_sc(rows_h, cols_h, vals_h, x0)

    out_f, out_ff = _mix_tc(x0, sums, concat_W, concat_b)
    s_f = out_f[:S_NUM]
    e_f = out_f[S_NUM:S_NUM + E_NUM]
    k_f = out_f[S_NUM + E_NUM:]
    s_ff = out_ff[:S_NUM]
    e_ff = out_ff[S_NUM:S_NUM + E_NUM]

    ps = _linear_tc(s_f, Wts, bts)
    pe = _linear_tc(e_f, Wte, bte)
    knowledge_ts = _linear_tc(k_f, Wtk, btk)

    ss = _nce_sum_tc(s_f, s_ff)
    se = _nce_sum_tc(e_f, e_ff)
    extra_loss = (SSL_WEIGHT * (ss[0, 0] / S_NUM + se[0, 0] / E_NUM)).astype(f32)

    sid32 = student_id.astype(jnp.int32)
    eid32 = exercise_id.astype(jnp.int32)
    disc8 = jnp.pad(disc_emb, ((0, 0), (0, 7)))
    student_ts, diff_ts = _gather_tables_sc(
        [ps, pe], [sid32.reshape(32, 16, 32), eid32.reshape(32, 16, 32)],
        [K_NUM, K_NUM], True, 32, 16)
    disc8_ts, ki_ts = _gather_tables_sc(
        [disc8, ki_emb], [eid32.reshape(32, 8, 64), eid32.reshape(32, 8, 64)],
        [8, D], False, 64, 8)
    disc_ts = disc8_ts[:, :1]

    return (student_ts, diff_ts, disc_ts, knowledge_ts, extra_loss, ki_ts)
